# baseline (speedup 1.0000x reference)
import functools

import jax
import jax.numpy as jnp
from jax import lax
from jax.experimental import pallas as pl
from jax.experimental.pallas import tpu as pltpu
from jax.experimental.pallas import tpu_sc as plsc

N = 100000
E = 1600000
TILE_TYPES = 32
EMB = 16
HID = 32
OUT = 16
MLP = 8
G = 64

NC, NS = 2, 16
NW = NC * NS
CHUNK = 128
NP = 100352
SUB_ROWS = NP // NS
ZCH = 128
NZ = SUB_ROWS // ZCH
RW = 400
CH = 8
EP = NW * RW * CHUNK
EPR = EP // CHUNK
W0 = 520
W1 = 280
NB0 = W0 // CH
NB1 = W1 // CH

BN = 5000
GRID = N // BN


def _deg_kernel(dstR):
    mesh = plsc.VectorSubcoreMesh(core_axis_name="c", subcore_axis_name="s")

    @functools.partial(
        pl.kernel, mesh=mesh,
        out_type=jax.ShapeDtypeStruct((NC, NP), jnp.float32),
        compiler_params=pltpu.CompilerParams(use_tc_tiling_on_sc=False),
        scratch_types=[
            pltpu.VMEM((CH, CHUNK), jnp.int32),
            pltpu.VMEM((CHUNK,), jnp.float32),
            pltpu.VMEM((896,), jnp.float32),
            pltpu.VMEM_SHARED((NP,), jnp.float32),
            pltpu.SemaphoreType.DMA,
        ],
    )
    def k(dst_h, out_h, didx, ones_v, zer_v, acc, sems):
        c = lax.axis_index("c")
        s = lax.axis_index("s")
        wid = c * NS + s

        for i in range(8):
            ones_v[pl.ds(i * 16, 16)] = jnp.ones((16,), jnp.float32)

        def zb(i, _):
            zer_v[pl.ds(i * 16, 16)] = jnp.zeros((16,), jnp.float32)
            return 0
        lax.fori_loop(0, 56, zb, 0)

        r0 = s * SUB_ROWS
        def zc(i, _):
            pltpu.sync_copy(zer_v, acc.at[pl.ds(r0 + i * 896, 896)])
            return 0
        lax.fori_loop(0, 7, zc, 0)
        plsc.subcore_barrier()

        row0 = c * (NS * W0) + s * jnp.where(c == 0, W0, W1)
        nb = jnp.where(c == 0, NB0, NB1)
        def blk(b, _):
            pltpu.sync_copy(dst_h.at[pl.ds(row0 + b * CH, CH)], didx)
            scat = [pltpu.async_copy(ones_v, acc.at[didx.at[j]], sems, add=True)
                    for j in range(CH)]
            for h in scat:
                h.wait()
            return 0
        lax.fori_loop(0, nb, blk, 0)
        plsc.subcore_barrier()

        wb = [pltpu.async_copy(acc.at[pl.ds(r0 + i * 896, 896)],
                               out_h.at[c, pl.ds(r0 + i * 896, 896)], sems)
              for i in range(7)]
        for h in wb:
            h.wait()

    return k(dstR)


def _edge_pass(table, srcR, dstR):
    mesh = plsc.VectorSubcoreMesh(core_axis_name="c", subcore_axis_name="s")

    @functools.partial(
        pl.kernel, mesh=mesh,
        out_type=jax.ShapeDtypeStruct((NC, NP, OUT), jnp.float32),
        compiler_params=pltpu.CompilerParams(use_tc_tiling_on_sc=False),
        scratch_types=[
            pltpu.VMEM((CH, CHUNK), jnp.int32),
            pltpu.VMEM((CH, CHUNK), jnp.int32),
            pltpu.VMEM((CH, CHUNK, OUT), jnp.float32),
            pltpu.VMEM((ZCH, OUT), jnp.float32),
            pltpu.VMEM_SHARED((NP, OUT), jnp.float32),
            pltpu.SemaphoreType.DMA,
            pltpu.SemaphoreType.DMA,
        ],
    )
    def k(tab_h, src_h, dst_h, out_h, sidx, didx, rows, zer_v, acc,
          semg, sems):
        c = lax.axis_index("c")
        s = lax.axis_index("s")
        wid = c * NS + s

        def zb(i, _):
            zer_v[i] = jnp.zeros((OUT,), jnp.float32)
            return 0
        lax.fori_loop(0, ZCH, zb, 0)

        r0 = s * SUB_ROWS
        def zc(i, _):
            pltpu.sync_copy(zer_v, acc.at[pl.ds(r0 + i * ZCH, ZCH)])
            return 0
        lax.fori_loop(0, NZ, zc, 0)
        plsc.subcore_barrier()

        row0 = c * (NS * W0) + s * jnp.where(c == 0, W0, W1)
        nb = jnp.where(c == 0, NB0, NB1)
        def blk(b, _):
            pltpu.sync_copy(src_h.at[pl.ds(row0 + b * CH, CH)], sidx)
            pltpu.sync_copy(dst_h.at[pl.ds(row0 + b * CH, CH)], didx)
            gath = [pltpu.async_copy(tab_h.at[sidx.at[j]], rows.at[j], semg)
                    for j in range(CH)]
            for h in gath:
                h.wait()
            scat = [pltpu.async_copy(rows.at[j], acc.at[didx.at[j]], sems,
                                     add=True)
                    for j in range(CH)]
            for h in scat:
                h.wait()
            return 0
        lax.fori_loop(0, nb, blk, 0)
        plsc.subcore_barrier()

        wb = [pltpu.async_copy(acc.at[pl.ds(r0 + i * 896, 896)],
                               out_h.at[c, pl.ds(r0 + i * 896, 896)], sems)
              for i in range(7)]
        for h in wb:
            h.wait()

    return k(table, srcR, dstR)


def _embed_kernel(x2d, deg0, deg1, emb_table):
    def body(x_r, d0_r, d1_r, emb_r, h0_r, dis_r):
        deg = d0_r[...] + d1_r[...] + 1.0
        dis = lax.rsqrt(deg)
        onehot = (x_r[...] == lax.broadcasted_iota(jnp.int32, (1, TILE_TYPES), 1)
                  ).astype(jnp.float32)
        h0 = jnp.dot(onehot, emb_r[...], preferred_element_type=jnp.float32)
        h0_r[...] = h0 * dis
        dis_r[...] = dis

    return pl.pallas_call(
        body,
        grid=(GRID,),
        in_specs=[
            pl.BlockSpec((BN, 1), lambda i: (i, 0)),
            pl.BlockSpec((BN, 1), lambda i: (i, 0)),
            pl.BlockSpec((BN, 1), lambda i: (i, 0)),
            pl.BlockSpec((TILE_TYPES, EMB), lambda i: (0, 0)),
        ],
        out_specs=[
            pl.BlockSpec((BN, EMB), lambda i: (i, 0)),
            pl.BlockSpec((BN, 1), lambda i: (i, 0)),
        ],
        out_shape=[
            jax.ShapeDtypeStruct((N, EMB), jnp.float32),
            jax.ShapeDtypeStruct((N, 1), jnp.float32),
        ],
    )(x2d, deg0, deg1, emb_table)


def _mid_kernel(acc, h0, dis2d, W1, b1, W2):
    def body(a_r, h0_r, dis_r, W1_r, b1_r, W2_r, g2_r):
        agg = a_r[0] + a_r[1] + h0_r[...]
        dis = dis_r[...]
        t = jnp.dot(agg, W1_r[...], preferred_element_type=jnp.float32)
        h1 = jnp.maximum(t * dis + b1_r[...], 0.0)
        g2 = jnp.dot(h1, W2_r[...], preferred_element_type=jnp.float32)
        g2_r[...] = g2 * dis

    return pl.pallas_call(
        body,
        grid=(GRID,),
        in_specs=[
            pl.BlockSpec((NC, BN, OUT), lambda i: (0, i, 0)),
            pl.BlockSpec((BN, OUT), lambda i: (i, 0)),
            pl.BlockSpec((BN, 1), lambda i: (i, 0)),
            pl.BlockSpec((EMB, HID), lambda i: (0, 0)),
            pl.BlockSpec((1, HID), lambda i: (0, 0)),
            pl.BlockSpec((HID, OUT), lambda i: (0, 0)),
        ],
        out_specs=pl.BlockSpec((BN, OUT), lambda i: (i, 0)),
        out_shape=jax.ShapeDtypeStruct((N, OUT), jnp.float32),
    )(acc, h0, dis2d, W1, b1, W2)


def _pool_kernel(acc, g2, dis2d, b2, batch2d, fcW, fcb):
    def body(a_r, g2_r, dis_r, b2_r, bat_r, fcW_r, fcb_r, out_r,
             psum, pcnt):
        i = pl.program_id(0)

        @pl.when(i == 0)
        def _init():
            psum[...] = jnp.zeros((G, OUT), jnp.float32)
            pcnt[...] = jnp.zeros((G, OUT), jnp.float32)

        out2 = dis_r[...] * (a_r[0] + a_r[1] + g2_r[...]) + b2_r[...]
        onehot = (bat_r[...] == lax.broadcasted_iota(jnp.int32, (1, G), 1)
                  ).astype(jnp.float32)
        psum[...] += lax.dot_general(
            onehot, out2, (((0,), (0,)), ((), ())),
            preferred_element_type=jnp.float32)
        pcnt[...] += lax.dot_general(
            onehot, jnp.ones((BN, OUT), jnp.float32), (((0,), (0,)), ((), ())),
            preferred_element_type=jnp.float32)

        @pl.when(i == GRID - 1)
        def _fin():
            pooled = psum[...] / jnp.maximum(pcnt[...], 1.0)
            out_r[...] = jnp.dot(pooled, fcW_r[...],
                                 preferred_element_type=jnp.float32) + fcb_r[...]

    return pl.pallas_call(
        body,
        grid=(GRID,),
        in_specs=[
            pl.BlockSpec((NC, BN, OUT), lambda i: (0, i, 0)),
            pl.BlockSpec((BN, OUT), lambda i: (i, 0)),
            pl.BlockSpec((BN, 1), lambda i: (i, 0)),
            pl.BlockSpec((1, OUT), lambda i: (0, 0)),
            pl.BlockSpec((BN, 1), lambda i: (i, 0)),
            pl.BlockSpec((OUT, MLP), lambda i: (0, 0)),
            pl.BlockSpec((1, MLP), lambda i: (0, 0)),
        ],
        out_specs=pl.BlockSpec((G, MLP), lambda i: (0, 0)),
        out_shape=jax.ShapeDtypeStruct((G, MLP), jnp.float32),
        scratch_shapes=[
            pltpu.VMEM((G, OUT), jnp.float32),
            pltpu.VMEM((G, OUT), jnp.float32),
        ],
    )(acc, g2, dis2d, b2, batch2d, fcW, fcb)


def kernel(x, edge_index, batch, emb_table, W1, b1, W2, b2, fcW, fcb):
    src = edge_index[0]
    dst = edge_index[1]
    pad = EP - E
    srcR = jnp.concatenate(
        [src, jnp.zeros((pad,), jnp.int32)]).reshape(EPR, CHUNK)
    dstR = jnp.concatenate(
        [dst, jnp.full((pad,), N, jnp.int32)]).reshape(EPR, CHUNK)

    degp = _deg_kernel(dstR)
    deg0 = degp[0, :N].reshape(N, 1)
    deg1 = degp[1, :N].reshape(N, 1)
    x2d = x.reshape(N, 1)

    h0, dis2d = _embed_kernel(x2d, deg0, deg1, emb_table)

    acc1 = _edge_pass(h0, srcR, dstR)
    g2 = _mid_kernel(acc1, h0, dis2d, W1, b1.reshape(1, HID), W2)

    acc2 = _edge_pass(g2, srcR, dstR)
    return _pool_kernel(acc2, g2, dis2d, b2.reshape(1, OUT),
                        batch.reshape(N, 1), fcW, fcb.reshape(1, MLP))


# speedup vs baseline: 1.0782x; 1.0782x over previous
import functools

import jax
import jax.numpy as jnp
from jax import lax
from jax.experimental import pallas as pl
from jax.experimental.pallas import tpu as pltpu
from jax.experimental.pallas import tpu_sc as plsc

N = 100000
E = 1600000
TILE_TYPES = 32
EMB = 16
HID = 32
OUT = 16
MLP = 8
G = 64

NC, NS = 2, 16
NW = NC * NS
CHUNK = 128
NP = 100352
SUB_ROWS = NP // NS
ZCH = 128
NZ = SUB_ROWS // ZCH
RW = 400
CH = 8
EP = NW * RW * CHUNK
EPR = EP // CHUNK
W0 = 624
W1 = 176
NB0 = W0 // CH
NB1 = W1 // CH

BN = 5000
GRID = N // BN


def _deg_kernel(dstR):
    mesh = plsc.VectorSubcoreMesh(core_axis_name="c", subcore_axis_name="s")

    @functools.partial(
        pl.kernel, mesh=mesh,
        out_type=jax.ShapeDtypeStruct((NC, NP), jnp.float32),
        compiler_params=pltpu.CompilerParams(use_tc_tiling_on_sc=False),
        scratch_types=[
            pltpu.VMEM((CH, CHUNK), jnp.int32),
            pltpu.VMEM((CHUNK,), jnp.float32),
            pltpu.VMEM((896,), jnp.float32),
            pltpu.VMEM_SHARED((NP,), jnp.float32),
            pltpu.SemaphoreType.DMA,
        ],
    )
    def k(dst_h, out_h, didx, ones_v, zer_v, acc, sems):
        c = lax.axis_index("c")
        s = lax.axis_index("s")
        wid = c * NS + s

        for i in range(8):
            ones_v[pl.ds(i * 16, 16)] = jnp.ones((16,), jnp.float32)

        def zb(i, _):
            zer_v[pl.ds(i * 16, 16)] = jnp.zeros((16,), jnp.float32)
            return 0
        lax.fori_loop(0, 56, zb, 0)

        r0 = s * SUB_ROWS
        def zc(i, _):
            pltpu.sync_copy(zer_v, acc.at[pl.ds(r0 + i * 896, 896)])
            return 0
        lax.fori_loop(0, 7, zc, 0)
        plsc.subcore_barrier()

        row0 = c * (NS * W0) + s * jnp.where(c == 0, W0, W1)
        nb = jnp.where(c == 0, NB0, NB1)
        def blk(b, _):
            pltpu.sync_copy(dst_h.at[pl.ds(row0 + b * CH, CH)], didx)
            scat = [pltpu.async_copy(ones_v, acc.at[didx.at[j]], sems, add=True)
                    for j in range(CH)]
            for h in scat:
                h.wait()
            return 0
        lax.fori_loop(0, nb, blk, 0)
        plsc.subcore_barrier()

        wb = [pltpu.async_copy(acc.at[pl.ds(r0 + i * 896, 896)],
                               out_h.at[c, pl.ds(r0 + i * 896, 896)], sems)
              for i in range(7)]
        for h in wb:
            h.wait()

    return k(dstR)


def _edge_pass(table, srcR, dstR):
    mesh = plsc.VectorSubcoreMesh(core_axis_name="c", subcore_axis_name="s")

    @functools.partial(
        pl.kernel, mesh=mesh,
        out_type=jax.ShapeDtypeStruct((NC, NP, OUT), jnp.float32),
        compiler_params=pltpu.CompilerParams(use_tc_tiling_on_sc=False),
        scratch_types=[
            pltpu.VMEM((CH, CHUNK), jnp.int32),
            pltpu.VMEM((CH, CHUNK), jnp.int32),
            pltpu.VMEM((CH, CHUNK, OUT), jnp.float32),
            pltpu.VMEM((ZCH, OUT), jnp.float32),
            pltpu.VMEM_SHARED((NP, OUT), jnp.float32),
            pltpu.SemaphoreType.DMA,
            pltpu.SemaphoreType.DMA,
        ],
    )
    def k(tab_h, src_h, dst_h, out_h, sidx, didx, rows, zer_v, acc,
          semg, sems):
        c = lax.axis_index("c")
        s = lax.axis_index("s")
        wid = c * NS + s

        def zb(i, _):
            zer_v[i] = jnp.zeros((OUT,), jnp.float32)
            return 0
        lax.fori_loop(0, ZCH, zb, 0)

        r0 = s * SUB_ROWS
        def zc(i, _):
            pltpu.sync_copy(zer_v, acc.at[pl.ds(r0 + i * ZCH, ZCH)])
            return 0
        lax.fori_loop(0, NZ, zc, 0)
        plsc.subcore_barrier()

        row0 = c * (NS * W0) + s * jnp.where(c == 0, W0, W1)
        nb = jnp.where(c == 0, NB0, NB1)
        def blk(b, _):
            pltpu.sync_copy(src_h.at[pl.ds(row0 + b * CH, CH)], sidx)
            pltpu.sync_copy(dst_h.at[pl.ds(row0 + b * CH, CH)], didx)
            gath = [pltpu.async_copy(tab_h.at[sidx.at[j]], rows.at[j], semg)
                    for j in range(CH)]
            for h in gath:
                h.wait()
            scat = [pltpu.async_copy(rows.at[j], acc.at[didx.at[j]], sems,
                                     add=True)
                    for j in range(CH)]
            for h in scat:
                h.wait()
            return 0
        lax.fori_loop(0, nb, blk, 0)
        plsc.subcore_barrier()

        wb = [pltpu.async_copy(acc.at[pl.ds(r0 + i * 896, 896)],
                               out_h.at[c, pl.ds(r0 + i * 896, 896)], sems)
              for i in range(7)]
        for h in wb:
            h.wait()

    return k(table, srcR, dstR)


def _embed_kernel(x2d, deg0, deg1, emb_table):
    def body(x_r, d0_r, d1_r, emb_r, h0_r, dis_r):
        deg = d0_r[...] + d1_r[...] + 1.0
        dis = lax.rsqrt(deg)
        onehot = (x_r[...] == lax.broadcasted_iota(jnp.int32, (1, TILE_TYPES), 1)
                  ).astype(jnp.float32)
        h0 = jnp.dot(onehot, emb_r[...], preferred_element_type=jnp.float32)
        h0_r[...] = h0 * dis
        dis_r[...] = dis

    return pl.pallas_call(
        body,
        grid=(GRID,),
        in_specs=[
            pl.BlockSpec((BN, 1), lambda i: (i, 0)),
            pl.BlockSpec((BN, 1), lambda i: (i, 0)),
            pl.BlockSpec((BN, 1), lambda i: (i, 0)),
            pl.BlockSpec((TILE_TYPES, EMB), lambda i: (0, 0)),
        ],
        out_specs=[
            pl.BlockSpec((BN, EMB), lambda i: (i, 0)),
            pl.BlockSpec((BN, 1), lambda i: (i, 0)),
        ],
        out_shape=[
            jax.ShapeDtypeStruct((N, EMB), jnp.float32),
            jax.ShapeDtypeStruct((N, 1), jnp.float32),
        ],
    )(x2d, deg0, deg1, emb_table)


def _mid_kernel(acc, h0, dis2d, W1, b1, W2):
    def body(a_r, h0_r, dis_r, W1_r, b1_r, W2_r, g2_r):
        agg = a_r[0] + a_r[1] + h0_r[...]
        dis = dis_r[...]
        t = jnp.dot(agg, W1_r[...], preferred_element_type=jnp.float32)
        h1 = jnp.maximum(t * dis + b1_r[...], 0.0)
        g2 = jnp.dot(h1, W2_r[...], preferred_element_type=jnp.float32)
        g2_r[...] = g2 * dis

    return pl.pallas_call(
        body,
        grid=(GRID,),
        in_specs=[
            pl.BlockSpec((NC, BN, OUT), lambda i: (0, i, 0)),
            pl.BlockSpec((BN, OUT), lambda i: (i, 0)),
            pl.BlockSpec((BN, 1), lambda i: (i, 0)),
            pl.BlockSpec((EMB, HID), lambda i: (0, 0)),
            pl.BlockSpec((1, HID), lambda i: (0, 0)),
            pl.BlockSpec((HID, OUT), lambda i: (0, 0)),
        ],
        out_specs=pl.BlockSpec((BN, OUT), lambda i: (i, 0)),
        out_shape=jax.ShapeDtypeStruct((N, OUT), jnp.float32),
    )(acc, h0, dis2d, W1, b1, W2)


def _pool_kernel(acc, g2, dis2d, b2, batch2d, fcW, fcb):
    def body(a_r, g2_r, dis_r, b2_r, bat_r, fcW_r, fcb_r, out_r,
             psum, pcnt):
        i = pl.program_id(0)

        @pl.when(i == 0)
        def _init():
            psum[...] = jnp.zeros((G, OUT), jnp.float32)
            pcnt[...] = jnp.zeros((G, OUT), jnp.float32)

        out2 = dis_r[...] * (a_r[0] + a_r[1] + g2_r[...]) + b2_r[...]
        onehot = (bat_r[...] == lax.broadcasted_iota(jnp.int32, (1, G), 1)
                  ).astype(jnp.float32)
        psum[...] += lax.dot_general(
            onehot, out2, (((0,), (0,)), ((), ())),
            preferred_element_type=jnp.float32)
        pcnt[...] += lax.dot_general(
            onehot, jnp.ones((BN, OUT), jnp.float32), (((0,), (0,)), ((), ())),
            preferred_element_type=jnp.float32)

        @pl.when(i == GRID - 1)
        def _fin():
            pooled = psum[...] / jnp.maximum(pcnt[...], 1.0)
            out_r[...] = jnp.dot(pooled, fcW_r[...],
                                 preferred_element_type=jnp.float32) + fcb_r[...]

    return pl.pallas_call(
        body,
        grid=(GRID,),
        in_specs=[
            pl.BlockSpec((NC, BN, OUT), lambda i: (0, i, 0)),
            pl.BlockSpec((BN, OUT), lambda i: (i, 0)),
            pl.BlockSpec((BN, 1), lambda i: (i, 0)),
            pl.BlockSpec((1, OUT), lambda i: (0, 0)),
            pl.BlockSpec((BN, 1), lambda i: (i, 0)),
            pl.BlockSpec((OUT, MLP), lambda i: (0, 0)),
            pl.BlockSpec((1, MLP), lambda i: (0, 0)),
        ],
        out_specs=pl.BlockSpec((G, MLP), lambda i: (0, 0)),
        out_shape=jax.ShapeDtypeStruct((G, MLP), jnp.float32),
        scratch_shapes=[
            pltpu.VMEM((G, OUT), jnp.float32),
            pltpu.VMEM((G, OUT), jnp.float32),
        ],
    )(acc, g2, dis2d, b2, batch2d, fcW, fcb)


def kernel(x, edge_index, batch, emb_table, W1, b1, W2, b2, fcW, fcb):
    src = edge_index[0]
    dst = edge_index[1]
    pad = EP - E
    srcR = jnp.concatenate(
        [src, jnp.zeros((pad,), jnp.int32)]).reshape(EPR, CHUNK)
    dstR = jnp.concatenate(
        [dst, jnp.full((pad,), N, jnp.int32)]).reshape(EPR, CHUNK)

    degp = _deg_kernel(dstR)
    deg0 = degp[0, :N].reshape(N, 1)
    deg1 = degp[1, :N].reshape(N, 1)
    x2d = x.reshape(N, 1)

    h0, dis2d = _embed_kernel(x2d, deg0, deg1, emb_table)

    acc1 = _edge_pass(h0, srcR, dstR)
    g2 = _mid_kernel(acc1, h0, dis2d, W1, b1.reshape(1, HID), W2)

    acc2 = _edge_pass(g2, srcR, dstR)
    return _pool_kernel(acc2, g2, dis2d, b2.reshape(1, OUT),
                        batch.reshape(N, 1), fcW, fcb.reshape(1, MLP))


# speedup vs baseline: 1.0995x; 1.0197x over previous
import functools

import jax
import jax.numpy as jnp
from jax import lax
from jax.experimental import pallas as pl
from jax.experimental.pallas import tpu as pltpu
from jax.experimental.pallas import tpu_sc as plsc

N = 100000
E = 1600000
TILE_TYPES = 32
EMB = 16
HID = 32
OUT = 16
MLP = 8
G = 64

NC, NS = 2, 16
NW = NC * NS
CHUNK = 128
NP = 100352
SUB_ROWS = NP // NS
ZCH = 128
NZ = SUB_ROWS // ZCH
RW = 400
CH = 8
EP = NW * RW * CHUNK
EPR = EP // CHUNK
W0 = 624
W1 = 176
NB0 = W0 // CH
NB1 = W1 // CH

BN = 5000
GRID = N // BN


def _deg_kernel(dstR):
    mesh = plsc.VectorSubcoreMesh(core_axis_name="c", subcore_axis_name="s")

    @functools.partial(
        pl.kernel, mesh=mesh,
        out_type=jax.ShapeDtypeStruct((NC, NP), jnp.float32),
        compiler_params=pltpu.CompilerParams(use_tc_tiling_on_sc=False),
        scratch_types=[
            pltpu.VMEM((CH, CHUNK), jnp.int32),
            pltpu.VMEM((CHUNK,), jnp.float32),
            pltpu.VMEM((896,), jnp.float32),
            pltpu.VMEM_SHARED((NP,), jnp.float32),
            pltpu.SemaphoreType.DMA,
        ],
    )
    def k(dst_h, out_h, didx, ones_v, zer_v, acc, sems):
        c = lax.axis_index("c")
        s = lax.axis_index("s")
        wid = c * NS + s

        for i in range(8):
            ones_v[pl.ds(i * 16, 16)] = jnp.ones((16,), jnp.float32)

        def zb(i, _):
            zer_v[pl.ds(i * 16, 16)] = jnp.zeros((16,), jnp.float32)
            return 0
        lax.fori_loop(0, 56, zb, 0)

        r0 = s * SUB_ROWS
        def zc(i, _):
            pltpu.sync_copy(zer_v, acc.at[pl.ds(r0 + i * 896, 896)])
            return 0
        lax.fori_loop(0, 7, zc, 0)
        plsc.subcore_barrier()

        row0 = c * (NS * W0) + s * jnp.where(c == 0, W0, W1)
        nb = jnp.where(c == 0, NB0, NB1)
        def blk(b, _):
            pltpu.sync_copy(dst_h.at[pl.ds(row0 + b * CH, CH)], didx)
            scat = [pltpu.async_copy(ones_v, acc.at[didx.at[j]], sems, add=True)
                    for j in range(CH)]
            for h in scat:
                h.wait()
            return 0
        lax.fori_loop(0, nb, blk, 0)
        plsc.subcore_barrier()

        wb = [pltpu.async_copy(acc.at[pl.ds(r0 + i * 896, 896)],
                               out_h.at[c, pl.ds(r0 + i * 896, 896)], sems)
              for i in range(7)]
        for h in wb:
            h.wait()

    return k(dstR)


def _edge_pass(table, srcR, dstR):
    mesh = plsc.VectorSubcoreMesh(core_axis_name="c", subcore_axis_name="s")

    @functools.partial(
        pl.kernel, mesh=mesh,
        out_type=jax.ShapeDtypeStruct((NC, NP, OUT), jnp.float32),
        compiler_params=pltpu.CompilerParams(use_tc_tiling_on_sc=False),
        scratch_types=[
            pltpu.VMEM((CH, CHUNK), jnp.int32),
            pltpu.VMEM((CH, CHUNK), jnp.int32),
            pltpu.VMEM((CH, CHUNK, OUT), jnp.float32),
            pltpu.VMEM((ZCH, OUT), jnp.float32),
            pltpu.VMEM_SHARED((NP, OUT), jnp.float32),
            pltpu.SemaphoreType.DMA,
            pltpu.SemaphoreType.DMA,
            pltpu.SemaphoreType.DMA,
        ],
    )
    def k(tab_h, src_h, dst_h, out_h, sidx, didx, rows, zer_v, acc,
          semg, semg2, sems):
        c = lax.axis_index("c")
        s = lax.axis_index("s")
        wid = c * NS + s

        def zb(i, _):
            zer_v[i] = jnp.zeros((OUT,), jnp.float32)
            return 0
        lax.fori_loop(0, ZCH, zb, 0)

        r0 = s * SUB_ROWS
        def zc(i, _):
            pltpu.sync_copy(zer_v, acc.at[pl.ds(r0 + i * ZCH, ZCH)])
            return 0
        lax.fori_loop(0, NZ, zc, 0)
        plsc.subcore_barrier()

        row0 = c * (NS * W0) + s * jnp.where(c == 0, W0, W1)
        nb = jnp.where(c == 0, NB0, NB1)
        H = CH // 2
        def blk(b, _):
            pltpu.sync_copy(src_h.at[pl.ds(row0 + b * CH, CH)], sidx)
            pltpu.sync_copy(dst_h.at[pl.ds(row0 + b * CH, CH)], didx)
            g0 = [pltpu.async_copy(tab_h.at[sidx.at[j]], rows.at[j], semg)
                  for j in range(H)]
            g1 = [pltpu.async_copy(tab_h.at[sidx.at[j]], rows.at[j], semg2)
                  for j in range(H, CH)]
            for h in g0:
                h.wait()
            s0 = [pltpu.async_copy(rows.at[j], acc.at[didx.at[j]], sems,
                                   add=True)
                  for j in range(H)]
            for h in g1:
                h.wait()
            s1 = [pltpu.async_copy(rows.at[j], acc.at[didx.at[j]], sems,
                                   add=True)
                  for j in range(H, CH)]
            for h in s0 + s1:
                h.wait()
            return 0
        lax.fori_loop(0, nb, blk, 0)
        plsc.subcore_barrier()

        wb = [pltpu.async_copy(acc.at[pl.ds(r0 + i * 896, 896)],
                               out_h.at[c, pl.ds(r0 + i * 896, 896)], sems)
              for i in range(7)]
        for h in wb:
            h.wait()

    return k(table, srcR, dstR)


def _embed_kernel(x2d, deg0, deg1, emb_table):
    def body(x_r, d0_r, d1_r, emb_r, h0_r, dis_r):
        deg = d0_r[...] + d1_r[...] + 1.0
        dis = lax.rsqrt(deg)
        onehot = (x_r[...] == lax.broadcasted_iota(jnp.int32, (1, TILE_TYPES), 1)
                  ).astype(jnp.float32)
        h0 = jnp.dot(onehot, emb_r[...], preferred_element_type=jnp.float32)
        h0_r[...] = h0 * dis
        dis_r[...] = dis

    return pl.pallas_call(
        body,
        grid=(GRID,),
        in_specs=[
            pl.BlockSpec((BN, 1), lambda i: (i, 0)),
            pl.BlockSpec((BN, 1), lambda i: (i, 0)),
            pl.BlockSpec((BN, 1), lambda i: (i, 0)),
            pl.BlockSpec((TILE_TYPES, EMB), lambda i: (0, 0)),
        ],
        out_specs=[
            pl.BlockSpec((BN, EMB), lambda i: (i, 0)),
            pl.BlockSpec((BN, 1), lambda i: (i, 0)),
        ],
        out_shape=[
            jax.ShapeDtypeStruct((N, EMB), jnp.float32),
            jax.ShapeDtypeStruct((N, 1), jnp.float32),
        ],
    )(x2d, deg0, deg1, emb_table)


def _mid_kernel(acc, h0, dis2d, W1, b1, W2):
    def body(a_r, h0_r, dis_r, W1_r, b1_r, W2_r, g2_r):
        agg = a_r[0] + a_r[1] + h0_r[...]
        dis = dis_r[...]
        t = jnp.dot(agg, W1_r[...], preferred_element_type=jnp.float32)
        h1 = jnp.maximum(t * dis + b1_r[...], 0.0)
        g2 = jnp.dot(h1, W2_r[...], preferred_element_type=jnp.float32)
        g2_r[...] = g2 * dis

    return pl.pallas_call(
        body,
        grid=(GRID,),
        in_specs=[
            pl.BlockSpec((NC, BN, OUT), lambda i: (0, i, 0)),
            pl.BlockSpec((BN, OUT), lambda i: (i, 0)),
            pl.BlockSpec((BN, 1), lambda i: (i, 0)),
            pl.BlockSpec((EMB, HID), lambda i: (0, 0)),
            pl.BlockSpec((1, HID), lambda i: (0, 0)),
            pl.BlockSpec((HID, OUT), lambda i: (0, 0)),
        ],
        out_specs=pl.BlockSpec((BN, OUT), lambda i: (i, 0)),
        out_shape=jax.ShapeDtypeStruct((N, OUT), jnp.float32),
    )(acc, h0, dis2d, W1, b1, W2)


def _pool_kernel(acc, g2, dis2d, b2, batch2d, fcW, fcb):
    def body(a_r, g2_r, dis_r, b2_r, bat_r, fcW_r, fcb_r, out_r,
             psum, pcnt):
        i = pl.program_id(0)

        @pl.when(i == 0)
        def _init():
            psum[...] = jnp.zeros((G, OUT), jnp.float32)
            pcnt[...] = jnp.zeros((G, OUT), jnp.float32)

        out2 = dis_r[...] * (a_r[0] + a_r[1] + g2_r[...]) + b2_r[...]
        onehot = (bat_r[...] == lax.broadcasted_iota(jnp.int32, (1, G), 1)
                  ).astype(jnp.float32)
        psum[...] += lax.dot_general(
            onehot, out2, (((0,), (0,)), ((), ())),
            preferred_element_type=jnp.float32)
        pcnt[...] += lax.dot_general(
            onehot, jnp.ones((BN, OUT), jnp.float32), (((0,), (0,)), ((), ())),
            preferred_element_type=jnp.float32)

        @pl.when(i == GRID - 1)
        def _fin():
            pooled = psum[...] / jnp.maximum(pcnt[...], 1.0)
            out_r[...] = jnp.dot(pooled, fcW_r[...],
                                 preferred_element_type=jnp.float32) + fcb_r[...]

    return pl.pallas_call(
        body,
        grid=(GRID,),
        in_specs=[
            pl.BlockSpec((NC, BN, OUT), lambda i: (0, i, 0)),
            pl.BlockSpec((BN, OUT), lambda i: (i, 0)),
            pl.BlockSpec((BN, 1), lambda i: (i, 0)),
            pl.BlockSpec((1, OUT), lambda i: (0, 0)),
            pl.BlockSpec((BN, 1), lambda i: (i, 0)),
            pl.BlockSpec((OUT, MLP), lambda i: (0, 0)),
            pl.BlockSpec((1, MLP), lambda i: (0, 0)),
        ],
        out_specs=pl.BlockSpec((G, MLP), lambda i: (0, 0)),
        out_shape=jax.ShapeDtypeStruct((G, MLP), jnp.float32),
        scratch_shapes=[
            pltpu.VMEM((G, OUT), jnp.float32),
            pltpu.VMEM((G, OUT), jnp.float32),
        ],
    )(acc, g2, dis2d, b2, batch2d, fcW, fcb)


def kernel(x, edge_index, batch, emb_table, W1, b1, W2, b2, fcW, fcb):
    src = edge_index[0]
    dst = edge_index[1]
    pad = EP - E
    srcR = jnp.concatenate(
        [src, jnp.zeros((pad,), jnp.int32)]).reshape(EPR, CHUNK)
    dstR = jnp.concatenate(
        [dst, jnp.full((pad,), N, jnp.int32)]).reshape(EPR, CHUNK)

    degp = _deg_kernel(dstR)
    deg0 = degp[0, :N].reshape(N, 1)
    deg1 = degp[1, :N].reshape(N, 1)
    x2d = x.reshape(N, 1)

    h0, dis2d = _embed_kernel(x2d, deg0, deg1, emb_table)

    acc1 = _edge_pass(h0, srcR, dstR)
    g2 = _mid_kernel(acc1, h0, dis2d, W1, b1.reshape(1, HID), W2)

    acc2 = _edge_pass(g2, srcR, dstR)
    return _pool_kernel(acc2, g2, dis2d, b2.reshape(1, OUT),
                        batch.reshape(N, 1), fcW, fcb.reshape(1, MLP))


# speedup vs baseline: 1.1210x; 1.0196x over previous
import functools

import jax
import jax.numpy as jnp
from jax import lax
from jax.experimental import pallas as pl
from jax.experimental.pallas import tpu as pltpu
from jax.experimental.pallas import tpu_sc as plsc

N = 100000
E = 1600000
TILE_TYPES = 32
EMB = 16
HID = 32
OUT = 16
MLP = 8
G = 64

NC, NS = 2, 16
NW = NC * NS
CHUNK = 128
NP = 100352
SUB_ROWS = NP // NS
ZCH = 128
NZ = SUB_ROWS // ZCH
RW = 400
CH = 8
EP = NW * RW * CHUNK
EPR = EP // CHUNK
W0 = 664
W1 = 136
NB0 = W0 // CH
NB1 = W1 // CH
D0 = 536
D1 = 264
ND0 = D0 // CH
ND1 = D1 // CH

BN = 5000
GRID = N // BN


def _deg_kernel(dstR):
    mesh = plsc.VectorSubcoreMesh(core_axis_name="c", subcore_axis_name="s")

    @functools.partial(
        pl.kernel, mesh=mesh,
        out_type=jax.ShapeDtypeStruct((NC, NP), jnp.float32),
        compiler_params=pltpu.CompilerParams(use_tc_tiling_on_sc=False),
        scratch_types=[
            pltpu.VMEM((CH, CHUNK), jnp.int32),
            pltpu.VMEM((CHUNK,), jnp.float32),
            pltpu.VMEM((896,), jnp.float32),
            pltpu.VMEM_SHARED((NP,), jnp.float32),
            pltpu.SemaphoreType.DMA,
        ],
    )
    def k(dst_h, out_h, didx, ones_v, zer_v, acc, sems):
        c = lax.axis_index("c")
        s = lax.axis_index("s")
        wid = c * NS + s

        for i in range(8):
            ones_v[pl.ds(i * 16, 16)] = jnp.ones((16,), jnp.float32)

        def zb(i, _):
            zer_v[pl.ds(i * 16, 16)] = jnp.zeros((16,), jnp.float32)
            return 0
        lax.fori_loop(0, 56, zb, 0)

        r0 = s * SUB_ROWS
        def zc(i, _):
            pltpu.sync_copy(zer_v, acc.at[pl.ds(r0 + i * 896, 896)])
            return 0
        lax.fori_loop(0, 7, zc, 0)
        plsc.subcore_barrier()

        row0 = c * (NS * D0) + s * jnp.where(c == 0, D0, D1)
        nb = jnp.where(c == 0, ND0, ND1)
        def blk(b, _):
            pltpu.sync_copy(dst_h.at[pl.ds(row0 + b * CH, CH)], didx)
            scat = [pltpu.async_copy(ones_v, acc.at[didx.at[j]], sems, add=True)
                    for j in range(CH)]
            for h in scat:
                h.wait()
            return 0
        lax.fori_loop(0, nb, blk, 0)
        plsc.subcore_barrier()

        wb = [pltpu.async_copy(acc.at[pl.ds(r0 + i * 896, 896)],
                               out_h.at[c, pl.ds(r0 + i * 896, 896)], sems)
              for i in range(7)]
        for h in wb:
            h.wait()

    return k(dstR)


def _edge_pass(table, srcR, dstR):
    mesh = plsc.VectorSubcoreMesh(core_axis_name="c", subcore_axis_name="s")

    @functools.partial(
        pl.kernel, mesh=mesh,
        out_type=jax.ShapeDtypeStruct((NC, NP, OUT), jnp.float32),
        compiler_params=pltpu.CompilerParams(use_tc_tiling_on_sc=False),
        scratch_types=[
            pltpu.VMEM((CH, CHUNK), jnp.int32),
            pltpu.VMEM((CH, CHUNK), jnp.int32),
            pltpu.VMEM((CH, CHUNK, OUT), jnp.float32),
            pltpu.VMEM((ZCH, OUT), jnp.float32),
            pltpu.VMEM_SHARED((NP, OUT), jnp.float32),
            pltpu.SemaphoreType.DMA,
            pltpu.SemaphoreType.DMA,
            pltpu.SemaphoreType.DMA,
        ],
    )
    def k(tab_h, src_h, dst_h, out_h, sidx, didx, rows, zer_v, acc,
          semg, semg2, sems):
        c = lax.axis_index("c")
        s = lax.axis_index("s")
        wid = c * NS + s

        def zb(i, _):
            zer_v[i] = jnp.zeros((OUT,), jnp.float32)
            return 0
        lax.fori_loop(0, ZCH, zb, 0)

        r0 = s * SUB_ROWS
        def zc(i, _):
            pltpu.sync_copy(zer_v, acc.at[pl.ds(r0 + i * ZCH, ZCH)])
            return 0
        lax.fori_loop(0, NZ, zc, 0)
        plsc.subcore_barrier()

        row0 = c * (NS * W0) + s * jnp.where(c == 0, W0, W1)
        nb = jnp.where(c == 0, NB0, NB1)
        H = CH // 2
        def blk(b, _):
            pltpu.sync_copy(src_h.at[pl.ds(row0 + b * CH, CH)], sidx)
            pltpu.sync_copy(dst_h.at[pl.ds(row0 + b * CH, CH)], didx)
            g0 = [pltpu.async_copy(tab_h.at[sidx.at[j]], rows.at[j], semg)
                  for j in range(H)]
            g1 = [pltpu.async_copy(tab_h.at[sidx.at[j]], rows.at[j], semg2)
                  for j in range(H, CH)]
            for h in g0:
                h.wait()
            s0 = [pltpu.async_copy(rows.at[j], acc.at[didx.at[j]], sems,
                                   add=True)
                  for j in range(H)]
            for h in g1:
                h.wait()
            s1 = [pltpu.async_copy(rows.at[j], acc.at[didx.at[j]], sems,
                                   add=True)
                  for j in range(H, CH)]
            for h in s0 + s1:
                h.wait()
            return 0
        lax.fori_loop(0, nb, blk, 0)
        plsc.subcore_barrier()

        wb = [pltpu.async_copy(acc.at[pl.ds(r0 + i * 896, 896)],
                               out_h.at[c, pl.ds(r0 + i * 896, 896)], sems)
              for i in range(7)]
        for h in wb:
            h.wait()

    return k(table, srcR, dstR)


def _embed_kernel(x2d, deg0, deg1, emb_table):
    def body(x_r, d0_r, d1_r, emb_r, h0_r, dis_r):
        deg = d0_r[...] + d1_r[...] + 1.0
        dis = lax.rsqrt(deg)
        onehot = (x_r[...] == lax.broadcasted_iota(jnp.int32, (1, TILE_TYPES), 1)
                  ).astype(jnp.float32)
        h0 = jnp.dot(onehot, emb_r[...], preferred_element_type=jnp.float32)
        h0_r[...] = h0 * dis
        dis_r[...] = dis

    return pl.pallas_call(
        body,
        grid=(GRID,),
        in_specs=[
            pl.BlockSpec((BN, 1), lambda i: (i, 0)),
            pl.BlockSpec((BN, 1), lambda i: (i, 0)),
            pl.BlockSpec((BN, 1), lambda i: (i, 0)),
            pl.BlockSpec((TILE_TYPES, EMB), lambda i: (0, 0)),
        ],
        out_specs=[
            pl.BlockSpec((BN, EMB), lambda i: (i, 0)),
            pl.BlockSpec((BN, 1), lambda i: (i, 0)),
        ],
        out_shape=[
            jax.ShapeDtypeStruct((N, EMB), jnp.float32),
            jax.ShapeDtypeStruct((N, 1), jnp.float32),
        ],
    )(x2d, deg0, deg1, emb_table)


def _mid_kernel(acc, h0, dis2d, W1, b1, W2):
    def body(a_r, h0_r, dis_r, W1_r, b1_r, W2_r, g2_r):
        agg = a_r[0] + a_r[1] + h0_r[...]
        dis = dis_r[...]
        t = jnp.dot(agg, W1_r[...], preferred_element_type=jnp.float32)
        h1 = jnp.maximum(t * dis + b1_r[...], 0.0)
        g2 = jnp.dot(h1, W2_r[...], preferred_element_type=jnp.float32)
        g2_r[...] = g2 * dis

    return pl.pallas_call(
        body,
        grid=(GRID,),
        in_specs=[
            pl.BlockSpec((NC, BN, OUT), lambda i: (0, i, 0)),
            pl.BlockSpec((BN, OUT), lambda i: (i, 0)),
            pl.BlockSpec((BN, 1), lambda i: (i, 0)),
            pl.BlockSpec((EMB, HID), lambda i: (0, 0)),
            pl.BlockSpec((1, HID), lambda i: (0, 0)),
            pl.BlockSpec((HID, OUT), lambda i: (0, 0)),
        ],
        out_specs=pl.BlockSpec((BN, OUT), lambda i: (i, 0)),
        out_shape=jax.ShapeDtypeStruct((N, OUT), jnp.float32),
    )(acc, h0, dis2d, W1, b1, W2)


def _pool_kernel(acc, g2, dis2d, b2, batch2d, fcW, fcb):
    def body(a_r, g2_r, dis_r, b2_r, bat_r, fcW_r, fcb_r, out_r,
             psum, pcnt):
        i = pl.program_id(0)

        @pl.when(i == 0)
        def _init():
            psum[...] = jnp.zeros((G, OUT), jnp.float32)
            pcnt[...] = jnp.zeros((G, OUT), jnp.float32)

        out2 = dis_r[...] * (a_r[0] + a_r[1] + g2_r[...]) + b2_r[...]
        onehot = (bat_r[...] == lax.broadcasted_iota(jnp.int32, (1, G), 1)
                  ).astype(jnp.float32)
        psum[...] += lax.dot_general(
            onehot, out2, (((0,), (0,)), ((), ())),
            preferred_element_type=jnp.float32)
        pcnt[...] += lax.dot_general(
            onehot, jnp.ones((BN, OUT), jnp.float32), (((0,), (0,)), ((), ())),
            preferred_element_type=jnp.float32)

        @pl.when(i == GRID - 1)
        def _fin():
            pooled = psum[...] / jnp.maximum(pcnt[...], 1.0)
            out_r[...] = jnp.dot(pooled, fcW_r[...],
                                 preferred_element_type=jnp.float32) + fcb_r[...]

    return pl.pallas_call(
        body,
        grid=(GRID,),
        in_specs=[
            pl.BlockSpec((NC, BN, OUT), lambda i: (0, i, 0)),
            pl.BlockSpec((BN, OUT), lambda i: (i, 0)),
            pl.BlockSpec((BN, 1), lambda i: (i, 0)),
            pl.BlockSpec((1, OUT), lambda i: (0, 0)),
            pl.BlockSpec((BN, 1), lambda i: (i, 0)),
            pl.BlockSpec((OUT, MLP), lambda i: (0, 0)),
            pl.BlockSpec((1, MLP), lambda i: (0, 0)),
        ],
        out_specs=pl.BlockSpec((G, MLP), lambda i: (0, 0)),
        out_shape=jax.ShapeDtypeStruct((G, MLP), jnp.float32),
        scratch_shapes=[
            pltpu.VMEM((G, OUT), jnp.float32),
            pltpu.VMEM((G, OUT), jnp.float32),
        ],
    )(acc, g2, dis2d, b2, batch2d, fcW, fcb)


def kernel(x, edge_index, batch, emb_table, W1, b1, W2, b2, fcW, fcb):
    src = edge_index[0]
    dst = edge_index[1]
    pad = EP - E
    srcR = jnp.concatenate(
        [src, jnp.zeros((pad,), jnp.int32)]).reshape(EPR, CHUNK)
    dstR = jnp.concatenate(
        [dst, jnp.full((pad,), N, jnp.int32)]).reshape(EPR, CHUNK)

    degp = _deg_kernel(dstR)
    deg0 = degp[0, :N].reshape(N, 1)
    deg1 = degp[1, :N].reshape(N, 1)
    x2d = x.reshape(N, 1)

    h0, dis2d = _embed_kernel(x2d, deg0, deg1, emb_table)

    acc1 = _edge_pass(h0, srcR, dstR)
    g2 = _mid_kernel(acc1, h0, dis2d, W1, b1.reshape(1, HID), W2)

    acc2 = _edge_pass(g2, srcR, dstR)
    return _pool_kernel(acc2, g2, dis2d, b2.reshape(1, OUT),
                        batch.reshape(N, 1), fcW, fcb.reshape(1, MLP))
